# Initial kernel scaffold; baseline (speedup 1.0000x reference)
import jax
import jax.numpy as jnp
from jax.experimental import pallas as pl


def kernel(wide_input, deep_input, table, W_wide, b_wide, W1, b1, W2, b2, W3, b3):
    raise NotImplementedError("write your pallas kernel here")


# speedup vs baseline: 4.0241x; 4.0241x over previous
import functools

import jax
import jax.numpy as jnp
from jax import lax
from jax.experimental import pallas as pl
from jax.experimental.pallas import tpu as pltpu
from jax.experimental.pallas import tpu_sc as plsc

_VOCAB = 1000000
_EMBED_DIM = 32
_BATCH = 4096
_N_FIELDS = 26
_WIDE_IN = 1024
_WIDE_OUT = 64
_H1 = 256
_H2 = 128
_DEEP_OUT = 64

_B_FLAT = _BATCH * _N_FIELDS

_NC = 2
_NS = 16
_NW = _NC * _NS
_B_PER_W = _B_FLAT // _NW


def _sc_gather(table, idx):
    mesh = plsc.VectorSubcoreMesh(core_axis_name="c", subcore_axis_name="s")

    @functools.partial(
        pl.kernel,
        mesh=mesh,
        out_type=jax.ShapeDtypeStruct((_B_FLAT, _EMBED_DIM), jnp.float32),
        compiler_params=pltpu.CompilerParams(use_tc_tiling_on_sc=False),
        scratch_types=[
            pltpu.VMEM((_B_PER_W,), jnp.int32),
            pltpu.VMEM((_B_PER_W, _EMBED_DIM), jnp.float32),
            pltpu.SemaphoreType.DMA,
        ],
    )
    def k(table_hbm, idx_hbm, out_hbm, idx_v, rows_v, sem):
        wid = lax.axis_index("s") * _NC + lax.axis_index("c")
        base = wid * _B_PER_W
        pltpu.sync_copy(idx_hbm.at[pl.ds(base, _B_PER_W)], idx_v)
        pltpu.async_copy(table_hbm.at[idx_v], rows_v, sem).wait()
        pltpu.sync_copy(rows_v, out_hbm.at[pl.ds(base, _B_PER_W)])

    return k(table, idx)


_WIDE_BLK = 512


def _wide_body(x_ref, w_ref, b_ref, o_ref):
    o_ref[...] = (
        jnp.dot(x_ref[...], w_ref[...], preferred_element_type=jnp.float32)
        + b_ref[...]
    )


def _tc_wide(wide_input, W_wide, b_wide):
    grid = (_BATCH // _WIDE_BLK,)
    return pl.pallas_call(
        _wide_body,
        grid=grid,
        in_specs=[
            pl.BlockSpec((_WIDE_BLK, _WIDE_IN), lambda i: (i, 0)),
            pl.BlockSpec((_WIDE_IN, _WIDE_OUT), lambda i: (0, 0)),
            pl.BlockSpec((1, _WIDE_OUT), lambda i: (0, 0)),
        ],
        out_specs=pl.BlockSpec((_WIDE_BLK, _WIDE_OUT), lambda i: (i, 0)),
        out_shape=jax.ShapeDtypeStruct((_BATCH, _WIDE_OUT), jnp.float32),
    )(wide_input, W_wide, b_wide.reshape(1, _WIDE_OUT))


_MLP_BLK = 2048


def _mlp_body(x_ref, w1_ref, b1_ref, w2_ref, b2_ref, w3_ref, b3_ref, o_ref):
    h = jnp.maximum(
        jnp.dot(x_ref[...], w1_ref[...], preferred_element_type=jnp.float32)
        + b1_ref[...],
        0.0,
    )
    h = jnp.maximum(
        jnp.dot(h, w2_ref[...], preferred_element_type=jnp.float32) + b2_ref[...],
        0.0,
    )
    o_ref[...] = (
        jnp.dot(h, w3_ref[...], preferred_element_type=jnp.float32) + b3_ref[...]
    )


def _tc_mlp(emb, W1, b1, W2, b2, W3, b3):
    grid = (_B_FLAT // _MLP_BLK,)
    return pl.pallas_call(
        _mlp_body,
        grid=grid,
        in_specs=[
            pl.BlockSpec((_MLP_BLK, _EMBED_DIM), lambda i: (i, 0)),
            pl.BlockSpec((_EMBED_DIM, _H1), lambda i: (0, 0)),
            pl.BlockSpec((1, _H1), lambda i: (0, 0)),
            pl.BlockSpec((_H1, _H2), lambda i: (0, 0)),
            pl.BlockSpec((1, _H2), lambda i: (0, 0)),
            pl.BlockSpec((_H2, _DEEP_OUT), lambda i: (0, 0)),
            pl.BlockSpec((1, _DEEP_OUT), lambda i: (0, 0)),
        ],
        out_specs=pl.BlockSpec((_MLP_BLK, _DEEP_OUT), lambda i: (i, 0)),
        out_shape=jax.ShapeDtypeStruct((_B_FLAT, _DEEP_OUT), jnp.float32),
    )(
        emb,
        W1,
        b1.reshape(1, _H1),
        W2,
        b2.reshape(1, _H2),
        W3,
        b3.reshape(1, _DEEP_OUT),
    )


def kernel(wide_input, deep_input, table, W_wide, b_wide, W1, b1, W2, b2, W3, b3):
    idx = deep_input.astype(jnp.int32).reshape(_B_FLAT)
    emb = _sc_gather(table, idx)
    wide_out = _tc_wide(wide_input, W_wide, b_wide)
    deep_flat = _tc_mlp(emb, W1, b1, W2, b2, W3, b3)
    deep_out = deep_flat.reshape(_BATCH, _N_FIELDS, _DEEP_OUT)
    return (wide_out, deep_out)


# speedup vs baseline: 4.3087x; 1.0707x over previous
import functools

import jax
import jax.numpy as jnp
from jax import lax
from jax.experimental import pallas as pl
from jax.experimental.pallas import tpu as pltpu
from jax.experimental.pallas import tpu_sc as plsc

_VOCAB = 1000000
_EMBED_DIM = 32
_BATCH = 4096
_N_FIELDS = 26
_WIDE_IN = 1024
_WIDE_OUT = 64
_H1 = 256
_H2 = 128
_DEEP_OUT = 64

_B_FLAT = _BATCH * _N_FIELDS
_B_PACK = _B_FLAT // 4

_NC = 2
_NS = 16
_NW = _NC * _NS
_B_PER_W = _B_FLAT // _NW
_P_PER_W = _B_PER_W // 4


def _sc_gather(table, idx):
    mesh = plsc.VectorSubcoreMesh(core_axis_name="c", subcore_axis_name="s")

    @functools.partial(
        pl.kernel,
        mesh=mesh,
        out_type=jax.ShapeDtypeStruct((_B_FLAT, _EMBED_DIM), jnp.float32),
        compiler_params=pltpu.CompilerParams(use_tc_tiling_on_sc=False),
        scratch_types=[
            pltpu.VMEM((_B_PER_W,), jnp.int32),
            pltpu.VMEM((_B_PER_W, _EMBED_DIM), jnp.float32),
            pltpu.SemaphoreType.DMA,
        ],
    )
    def k(table_hbm, idx_hbm, out_hbm, idx_v, rows_v, sem):
        wid = lax.axis_index("s") * _NC + lax.axis_index("c")
        pltpu.sync_copy(idx_hbm.at[pl.ds(wid * _B_PER_W, _B_PER_W)], idx_v)
        pltpu.async_copy(table_hbm.at[idx_v], rows_v, sem).wait()
        pltpu.sync_copy(rows_v, out_hbm.at[pl.ds(wid * _B_PER_W, _B_PER_W)])

    return k(table, idx)


_WIDE_BLK = 512


def _wide_body(x_ref, w_ref, b_ref, o_ref):
    y = (
        jnp.dot(x_ref[...], w_ref[...], preferred_element_type=jnp.float32)
        + b_ref[...]
    )
    o_ref[...] = y.T


def _tc_wide(wide_input, W_wide, b_wide):
    grid = (_BATCH // _WIDE_BLK,)
    return pl.pallas_call(
        _wide_body,
        grid=grid,
        in_specs=[
            pl.BlockSpec((_WIDE_BLK, _WIDE_IN), lambda i: (i, 0)),
            pl.BlockSpec((_WIDE_IN, _WIDE_OUT), lambda i: (0, 0)),
            pl.BlockSpec((1, _WIDE_OUT), lambda i: (0, 0)),
        ],
        out_specs=pl.BlockSpec((_WIDE_OUT, _WIDE_BLK), lambda i: (0, i)),
        out_shape=jax.ShapeDtypeStruct((_WIDE_OUT, _BATCH), jnp.float32),
    )(wide_input, W_wide, b_wide.reshape(1, _WIDE_OUT))


_MLP_BLK = 1024
_MLP_PBLK = _MLP_BLK // 4
_MLP_J = _BATCH // _MLP_BLK


def _mlp_body(x_ref, w1_ref, b1_ref, w2_ref, b2_ref, w3_ref, b3_ref, o_ref):
    ys = []
    for k in range(4):
        x = x_ref[:, k * _EMBED_DIM : (k + 1) * _EMBED_DIM]
        h = jnp.maximum(
            jnp.dot(x, w1_ref[...], preferred_element_type=jnp.float32)
            + b1_ref[...],
            0.0,
        )
        h = jnp.maximum(
            jnp.dot(h, w2_ref[...], preferred_element_type=jnp.float32)
            + b2_ref[...],
            0.0,
        )
        ys.append(
            jnp.dot(h, w3_ref[...], preferred_element_type=jnp.float32)
            + b3_ref[...]
        )
    y = jnp.stack(ys, axis=1).reshape(_MLP_BLK, _DEEP_OUT)
    o_ref[...] = y.T[None]


def _tc_mlp(emb_pack, W1, b1, W2, b2, W3, b3):
    grid = (_N_FIELDS, _MLP_J)
    return pl.pallas_call(
        _mlp_body,
        grid=grid,
        in_specs=[
            pl.BlockSpec((_MLP_PBLK, 4 * _EMBED_DIM), lambda f, j: (f * _MLP_J + j, 0)),
            pl.BlockSpec((_EMBED_DIM, _H1), lambda f, j: (0, 0)),
            pl.BlockSpec((1, _H1), lambda f, j: (0, 0)),
            pl.BlockSpec((_H1, _H2), lambda f, j: (0, 0)),
            pl.BlockSpec((1, _H2), lambda f, j: (0, 0)),
            pl.BlockSpec((_H2, _DEEP_OUT), lambda f, j: (0, 0)),
            pl.BlockSpec((1, _DEEP_OUT), lambda f, j: (0, 0)),
        ],
        out_specs=pl.BlockSpec((1, _DEEP_OUT, _MLP_BLK), lambda f, j: (f, 0, j)),
        out_shape=jax.ShapeDtypeStruct((_N_FIELDS, _DEEP_OUT, _BATCH), jnp.float32),
    )(
        emb_pack,
        W1,
        b1.reshape(1, _H1),
        W2,
        b2.reshape(1, _H2),
        W3,
        b3.reshape(1, _DEEP_OUT),
    )


def kernel(wide_input, deep_input, table, W_wide, b_wide, W1, b1, W2, b2, W3, b3):
    idx = deep_input.astype(jnp.int32).T.reshape(_B_FLAT)
    emb_pack = _sc_gather(table, idx).reshape(_B_PACK, 4 * _EMBED_DIM)
    wide_t = _tc_wide(wide_input, W_wide, b_wide)
    deep_t = _tc_mlp(emb_pack, W1, b1, W2, b2, W3, b3)
    wide_out = wide_t.T
    deep_out = jnp.transpose(deep_t, (2, 0, 1))
    return (wide_out, deep_out)


# speedup vs baseline: 5.4439x; 1.2635x over previous
import functools

import jax
import jax.numpy as jnp
from jax import lax
from jax.experimental import pallas as pl
from jax.experimental.pallas import tpu as pltpu
from jax.experimental.pallas import tpu_sc as plsc

_VOCAB = 1000000
_EMBED_DIM = 32
_BATCH = 4096
_N_FIELDS = 26
_WIDE_IN = 1024
_WIDE_OUT = 64
_H1 = 256
_H2 = 128
_DEEP_OUT = 64

_B_FLAT = _BATCH * _N_FIELDS
_B_PACK = _B_FLAT // 4

_NC = 2
_NS = 16
_NW = _NC * _NS
_B_PER_W = _B_FLAT // _NW
_P_PER_W = _B_PER_W // 4


_TR_BLKV = 16384


def _transpose_body(x_ref, o_ref):
    xt = x_ref[...].T
    xt4 = xt.reshape(_TR_BLKV // 4, 4, _EMBED_DIM)
    o_ref[...] = jnp.concatenate(
        [xt4[:, 0, :], xt4[:, 1, :], xt4[:, 2, :], xt4[:, 3, :]], axis=1
    )


def _tc_table_pack(table_t):
    grid = (pl.cdiv(_VOCAB, _TR_BLKV),)
    return pl.pallas_call(
        _transpose_body,
        grid=grid,
        in_specs=[pl.BlockSpec((_EMBED_DIM, _TR_BLKV), lambda i: (0, i))],
        out_specs=pl.BlockSpec((_TR_BLKV // 4, 4 * _EMBED_DIM), lambda i: (i, 0)),
        out_shape=jax.ShapeDtypeStruct((_VOCAB // 4, 4 * _EMBED_DIM), jnp.float32),
    )(table_t)


def _sc_gather(table, idx):
    mesh = plsc.VectorSubcoreMesh(core_axis_name="c", subcore_axis_name="s")

    @functools.partial(
        pl.kernel,
        mesh=mesh,
        out_type=jax.ShapeDtypeStruct((_B_FLAT, _EMBED_DIM), jnp.float32),
        compiler_params=pltpu.CompilerParams(use_tc_tiling_on_sc=False),
        scratch_types=[
            pltpu.VMEM((_B_PER_W,), jnp.int32),
            pltpu.VMEM((_B_PER_W, _EMBED_DIM), jnp.float32),
            pltpu.SemaphoreType.DMA,
        ],
    )
    def k(table_hbm, idx_hbm, out_hbm, idx_v, rows_v, sem):
        wid = lax.axis_index("s") * _NC + lax.axis_index("c")
        pltpu.sync_copy(idx_hbm.at[pl.ds(wid * _B_PER_W, _B_PER_W)], idx_v)
        pltpu.async_copy(table_hbm.at[idx_v], rows_v, sem).wait()
        pltpu.sync_copy(rows_v, out_hbm.at[pl.ds(wid * _B_PER_W, _B_PER_W)])

    return k(table, idx)


_WIDE_BLK = 512


def _wide_body(x_ref, w_ref, b_ref, o_ref):
    y = (
        jnp.dot(x_ref[...], w_ref[...], preferred_element_type=jnp.float32)
        + b_ref[...]
    )
    o_ref[...] = y.T


def _tc_wide(wide_input, W_wide, b_wide):
    grid = (_BATCH // _WIDE_BLK,)
    return pl.pallas_call(
        _wide_body,
        grid=grid,
        in_specs=[
            pl.BlockSpec((_WIDE_BLK, _WIDE_IN), lambda i: (i, 0)),
            pl.BlockSpec((_WIDE_IN, _WIDE_OUT), lambda i: (0, 0)),
            pl.BlockSpec((1, _WIDE_OUT), lambda i: (0, 0)),
        ],
        out_specs=pl.BlockSpec((_WIDE_OUT, _WIDE_BLK), lambda i: (0, i)),
        out_shape=jax.ShapeDtypeStruct((_WIDE_OUT, _BATCH), jnp.float32),
    )(wide_input, W_wide, b_wide.reshape(1, _WIDE_OUT))


_MLP_BLK = 2048
_MLP_PBLK = _MLP_BLK // 4
_MLP_J = _BATCH // _MLP_BLK


def _mlp_body(x_ref, w1_ref, b1_ref, w2_ref, b2_ref, w3_ref, b3_ref, o_ref):
    w1 = w1_ref[...].astype(jnp.bfloat16)
    w2 = w2_ref[...].astype(jnp.bfloat16)
    w3 = w3_ref[...].astype(jnp.bfloat16)
    ys = []
    for k in range(4):
        x = x_ref[:, k * _EMBED_DIM : (k + 1) * _EMBED_DIM].astype(jnp.bfloat16)
        h = jnp.maximum(
            jnp.dot(x, w1, preferred_element_type=jnp.float32) + b1_ref[...],
            0.0,
        ).astype(jnp.bfloat16)
        h = jnp.maximum(
            jnp.dot(h, w2, preferred_element_type=jnp.float32) + b2_ref[...],
            0.0,
        ).astype(jnp.bfloat16)
        ys.append(
            jnp.dot(h, w3, preferred_element_type=jnp.float32) + b3_ref[...]
        )
    y = jnp.stack(ys, axis=1).reshape(_MLP_BLK, _DEEP_OUT)
    o_ref[...] = y.T[None]


def _tc_mlp(emb_pack, W1, b1, W2, b2, W3, b3):
    grid = (_N_FIELDS, _MLP_J)
    return pl.pallas_call(
        _mlp_body,
        grid=grid,
        in_specs=[
            pl.BlockSpec((_MLP_PBLK, 4 * _EMBED_DIM), lambda f, j: (f * _MLP_J + j, 0)),
            pl.BlockSpec((_EMBED_DIM, _H1), lambda f, j: (0, 0)),
            pl.BlockSpec((1, _H1), lambda f, j: (0, 0)),
            pl.BlockSpec((_H1, _H2), lambda f, j: (0, 0)),
            pl.BlockSpec((1, _H2), lambda f, j: (0, 0)),
            pl.BlockSpec((_H2, _DEEP_OUT), lambda f, j: (0, 0)),
            pl.BlockSpec((1, _DEEP_OUT), lambda f, j: (0, 0)),
        ],
        out_specs=pl.BlockSpec((1, _DEEP_OUT, _MLP_BLK), lambda f, j: (f, 0, j)),
        out_shape=jax.ShapeDtypeStruct((_N_FIELDS, _DEEP_OUT, _BATCH), jnp.float32),
    )(
        emb_pack,
        W1,
        b1.reshape(1, _H1),
        W2,
        b2.reshape(1, _H2),
        W3,
        b3.reshape(1, _DEEP_OUT),
    )


def kernel(wide_input, deep_input, table, W_wide, b_wide, W1, b1, W2, b2, W3, b3):
    idx = deep_input.astype(jnp.int32).T.reshape(_B_FLAT)
    table_rm = _tc_table_pack(table.T).reshape(_VOCAB, _EMBED_DIM)
    emb_pack = _sc_gather(table_rm, idx).reshape(_B_PACK, 4 * _EMBED_DIM)
    wide_t = _tc_wide(wide_input, W_wide, b_wide)
    deep_t = _tc_mlp(emb_pack, W1, b1, W2, b2, W3, b3)
    wide_out = wide_t.T
    deep_out = jnp.transpose(deep_t, (2, 0, 1))
    return (wide_out, deep_out)
